# baseline (speedup 1.0000x reference)
import functools

import jax
import jax.numpy as jnp
from jax import lax
from jax.experimental import pallas as pl
from jax.experimental.pallas import tpu as pltpu
from jax.experimental.pallas import tpu_sc as plsc

_B = 16384
_D = 32
_NC = 2
_NS = 16
_NW = _NC * _NS
_BPW = _B // _NW
_L = 16


def _mf_body(users_hbm, items_hbm, uemb_hbm, iemb_hbm, ubias_hbm, ibias_hbm,
             out_hbm, uidx_v, iidx_v, urows_v, irows_v, ub_v, ib_v, out_v,
             sem):
    wid = lax.axis_index("s") * _NC + lax.axis_index("c")
    base = wid * _BPW

    pltpu.sync_copy(users_hbm.at[pl.ds(base, _BPW)], uidx_v)
    pltpu.sync_copy(items_hbm.at[pl.ds(base, _BPW)], iidx_v)

    cp_u = pltpu.async_copy(uemb_hbm.at[uidx_v], urows_v, sem)
    cp_i = pltpu.async_copy(iemb_hbm.at[iidx_v], irows_v, sem)
    cp_ub = pltpu.async_copy(ubias_hbm.at[uidx_v], ub_v, sem)
    cp_ib = pltpu.async_copy(ibias_hbm.at[iidx_v], ib_v, sem)
    cp_u.wait()
    cp_i.wait()
    cp_ub.wait()
    cp_ib.wait()

    lanes = lax.iota(jnp.int32, _L)

    def g_step(g, carry):
        rows = lanes + g * _L
        acc = ub_v[pl.ds(g * _L, _L)] + ib_v[pl.ds(g * _L, _L)]
        for j in range(_D):
            cols = jnp.full((_L,), j, jnp.int32)
            acc = acc + (plsc.load_gather(urows_v, [rows, cols]) *
                         plsc.load_gather(irows_v, [rows, cols]))
        out_v[pl.ds(g * _L, _L)] = acc
        return carry

    lax.fori_loop(0, _BPW // _L, g_step, 0)

    pltpu.sync_copy(out_v, out_hbm.at[pl.ds(base, _BPW)])


@functools.partial(jax.jit, static_argnums=())
def _mf_call(users, items, user_emb, item_emb, ubias, ibias):
    mesh = plsc.VectorSubcoreMesh(core_axis_name="c", subcore_axis_name="s")
    k = functools.partial(
        pl.kernel,
        out_type=jax.ShapeDtypeStruct((_B,), jnp.float32),
        mesh=mesh,
        compiler_params=pltpu.CompilerParams(
            use_tc_tiling_on_sc=False, needs_layout_passes=False),
        scratch_types=[
            pltpu.VMEM((_BPW,), jnp.int32),
            pltpu.VMEM((_BPW,), jnp.int32),
            pltpu.VMEM((_BPW, _D), jnp.float32),
            pltpu.VMEM((_BPW, _D), jnp.float32),
            pltpu.VMEM((_BPW,), jnp.float32),
            pltpu.VMEM((_BPW,), jnp.float32),
            pltpu.VMEM((_BPW,), jnp.float32),
            pltpu.SemaphoreType.DMA,
        ],
    )(_mf_body)
    return k(users, items, user_emb, item_emb, ubias, ibias)


def kernel(users, items, user_emb, item_emb, user_bias_tab, item_bias_tab):
    users = users.astype(jnp.int32)
    items = items.astype(jnp.int32)
    ubias = user_bias_tab.reshape(-1)
    ibias = item_bias_tab.reshape(-1)
    out = _mf_call(users, items, user_emb, item_emb, ubias, ibias)
    return out.reshape(_B, 1)
